# Initial kernel scaffold; baseline (speedup 1.0000x reference)
import jax
import jax.numpy as jnp
from jax.experimental import pallas as pl


def kernel(x, concat_feats, srt_concat_batch, end_concat_batch, num_concat_feats, batch, W_init, b_init, W_h1, b_h1, W_mol, b_mol, W_out, b_out):
    raise NotImplementedError("write your pallas kernel here")


# speedup vs baseline: 2.5130x; 2.5130x over previous
import functools

import jax
import jax.numpy as jnp
from jax import lax
from jax.experimental import pallas as pl
from jax.experimental.pallas import tpu as pltpu
from jax.experimental.pallas import tpu_sc as plsc

N = 10000
N_PAD = 10240
E = 320000
H = 128
D_EDGE = 16
G = 256

NC = 2
NS = 16
NW = NC * NS

_MESH = plsc.VectorSubcoreMesh(
    core_axis_name="c", subcore_axis_name="s", num_cores=NC, num_subcores=NS)


_BE = 2560


def _h0_body(f_ref, w_ref, b_ref, o_ref):
    v = jnp.dot(f_ref[...], w_ref[...], preferred_element_type=jnp.float32)
    v = v + b_ref[...]
    o_ref[...] = jnp.maximum(v, 0.01 * v)


def _tc_h0(feats, w_t, b_row):
    return pl.pallas_call(
        _h0_body,
        grid=(E // _BE,),
        in_specs=[
            pl.BlockSpec((_BE, D_EDGE), lambda i: (i, 0)),
            pl.BlockSpec((D_EDGE, H), lambda i: (0, 0)),
            pl.BlockSpec((1, H), lambda i: (0, 0)),
        ],
        out_specs=pl.BlockSpec((_BE, H), lambda i: (i, 0)),
        out_shape=jax.ShapeDtypeStruct((E, H), jnp.float32),
    )(feats, w_t, b_row)


_SCH = 256
_SC_NCH = E // _SCH
_SC_IT = -(-_SC_NCH // NW)
_STRIPE = N_PAD // NS


def _scatter_body(h_hbm, idx_hbm, z_hbm, out_hbm, acc, rows, idxv):
    cid = lax.axis_index("c")
    sid = lax.axis_index("s")
    wid = sid * NC + cid
    stripe = sid * _STRIPE
    pltpu.sync_copy(z_hbm.at[pl.ds(stripe, _STRIPE)],
                    acc.at[pl.ds(stripe, _STRIPE)])
    plsc.subcore_barrier()

    def chunk(i, _):
        c = wid + i * NW

        @pl.when(c < _SC_NCH)
        def _():
            pltpu.sync_copy(idx_hbm.at[pl.ds(c * 8, 8)], idxv)
            pltpu.sync_copy(h_hbm.at[pl.ds(c * _SCH, _SCH)], rows)
            for j in range(_SCH // 128):
                pltpu.sync_copy(rows.at[pl.ds(j * 128, 128)],
                                acc.at[idxv.at[j]], add=True)
        return 0

    lax.fori_loop(0, _SC_IT, chunk, 0)
    plsc.subcore_barrier()
    pltpu.sync_copy(acc.at[pl.ds(stripe, _STRIPE)],
                    out_hbm.at[cid, pl.ds(stripe, _STRIPE)])


@functools.partial(
    pl.kernel,
    out_type=jax.ShapeDtypeStruct((NC, N_PAD, H), jnp.float32),
    mesh=_MESH,
    scratch_types=[
        pltpu.VMEM_SHARED((N_PAD, H), jnp.float32),
        pltpu.VMEM((_SCH, H), jnp.float32),
        pltpu.VMEM((8, 128), jnp.int32),
    ],
)
def _sc_scatter(h_hbm, idx_hbm, z_hbm, out_hbm, acc, rows, idxv):
    _scatter_body(h_hbm, idx_hbm, z_hbm, out_hbm, acc, rows, idxv)


_BN = 2000


def _ab_body(s_ref, h_ref, w_ref, b_ref, a_ref, b_out_ref):
    s = s_ref[0] + s_ref[1]
    a_ref[...] = jnp.dot(s, w_ref[...], preferred_element_type=jnp.float32) \
        + b_ref[...]
    b_out_ref[...] = jnp.dot(h_ref[...], w_ref[...],
                             preferred_element_type=jnp.float32)


def _tc_ab(s_parts, h_n, w_t, b_row):
    return pl.pallas_call(
        _ab_body,
        grid=(N // _BN,),
        in_specs=[
            pl.BlockSpec((NC, _BN, H), lambda i: (0, i, 0)),
            pl.BlockSpec((_BN, H), lambda i: (i, 0)),
            pl.BlockSpec((H, H), lambda i: (0, 0)),
            pl.BlockSpec((1, H), lambda i: (0, 0)),
        ],
        out_specs=[
            pl.BlockSpec((_BN, H), lambda i: (i, 0)),
            pl.BlockSpec((_BN, H), lambda i: (i, 0)),
        ],
        out_shape=[
            jax.ShapeDtypeStruct((N, H), jnp.float32),
            jax.ShapeDtypeStruct((N, H), jnp.float32),
        ],
    )(s_parts, h_n, w_t, b_row)


_GCH = 256
_GC_NCH = E // _GCH
_GC_IT = -(-_GC_NCH // NW)


def _gather_body(h_hbm, a_hbm, b_hbm, srt_hbm, end_hbm, out_hbm,
                 h_rows, a_rows, b_rows, sidx, eidx, sem):
    cid = lax.axis_index("c")
    sid = lax.axis_index("s")
    wid = sid * NC + cid

    def chunk(i, _):
        c = wid + i * NW

        @pl.when(c < _GC_NCH)
        def _():
            k = _GCH // 128
            pltpu.sync_copy(srt_hbm.at[pl.ds(c * _GCH, _GCH)], sidx)
            pltpu.sync_copy(end_hbm.at[pl.ds(c * _GCH, _GCH)], eidx)
            cps = []
            for j in range(k):
                cps.append(pltpu.async_copy(
                    a_hbm.at[sidx.at[pl.ds(j * 128, 128)]],
                    a_rows.at[pl.ds(j * 128, 128)], sem))
                cps.append(pltpu.async_copy(
                    b_hbm.at[eidx.at[pl.ds(j * 128, 128)]],
                    b_rows.at[pl.ds(j * 128, 128)], sem))
            pltpu.sync_copy(h_hbm.at[pl.ds(c * _GCH, _GCH)], h_rows)
            for cp in cps:
                cp.wait()

            def row(r, _):
                for kk in range(H // 16):
                    sl = pl.ds(kk * 16, 16)
                    v = h_rows[r, sl] + a_rows[r, sl] - b_rows[r, sl]
                    h_rows[r, sl] = jnp.maximum(v, 0.01 * v)
                return 0

            lax.fori_loop(0, _GCH, row, 0)
            pltpu.sync_copy(h_rows, out_hbm.at[pl.ds(c * _GCH, _GCH)])
        return 0

    lax.fori_loop(0, _GC_IT, chunk, 0)


@functools.partial(
    pl.kernel,
    out_type=jax.ShapeDtypeStruct((E, H), jnp.float32),
    mesh=_MESH,
    scratch_types=[
        pltpu.VMEM((_GCH, H), jnp.float32),
        pltpu.VMEM((_GCH, H), jnp.float32),
        pltpu.VMEM((_GCH, H), jnp.float32),
        pltpu.VMEM((_GCH,), jnp.int32),
        pltpu.VMEM((_GCH,), jnp.int32),
        pltpu.SemaphoreType.DMA,
    ],
)
def _sc_gather_combine(h_hbm, a_hbm, b_hbm, srt_hbm, end_hbm, out_hbm,
                       h_rows, a_rows, b_rows, sidx, eidx, sem):
    _gather_body(h_hbm, a_hbm, b_hbm, srt_hbm, end_hbm, out_hbm,
                 h_rows, a_rows, b_rows, sidx, eidx, sem)


def _final_body(hn_ref, batch_ref, wm_ref, bm_ref, wo_ref, bo_ref,
                ae_ref, out_ref, hg_acc):
    i = pl.program_id(0)
    hn = hn_ref[0] + hn_ref[1]
    ae = jnp.dot(hn, wm_ref[...], preferred_element_type=jnp.float32) \
        + bm_ref[...]
    ae = jnp.maximum(ae, 0.01 * ae)
    ae_ref[...] = ae
    nrm = jnp.sqrt(jnp.sum(ae * ae, axis=1, keepdims=True))
    an = ae / jnp.maximum(nrm, 1e-12)
    bvec = batch_ref[0, 0, :]
    onehot = (bvec[:, None]
              == lax.broadcasted_iota(jnp.int32, (_BN, G), 1)).astype(
                  jnp.float32)
    contrib = lax.dot_general(onehot, an, (((0,), (0,)), ((), ())),
                              preferred_element_type=jnp.float32)

    @pl.when(i == 0)
    def _():
        hg_acc[...] = jnp.zeros_like(hg_acc)

    hg_acc[...] += contrib

    @pl.when(i == N // _BN - 1)
    def _():
        out_ref[...] = jnp.dot(hg_acc[...], wo_ref[...],
                               preferred_element_type=jnp.float32) \
            + bo_ref[...]


def _tc_final(hn_parts, batch3, wm_t, bm_row, wo_t, bo_row):
    return pl.pallas_call(
        _final_body,
        grid=(N // _BN,),
        in_specs=[
            pl.BlockSpec((NC, _BN, H), lambda i: (0, i, 0)),
            pl.BlockSpec((1, 1, _BN), lambda i: (i, 0, 0)),
            pl.BlockSpec((H, H), lambda i: (0, 0)),
            pl.BlockSpec((1, H), lambda i: (0, 0)),
            pl.BlockSpec((H, H), lambda i: (0, 0)),
            pl.BlockSpec((1, H), lambda i: (0, 0)),
        ],
        out_specs=[
            pl.BlockSpec((_BN, H), lambda i: (i, 0)),
            pl.BlockSpec((G, H), lambda i: (0, 0)),
        ],
        out_shape=[
            jax.ShapeDtypeStruct((N, H), jnp.float32),
            jax.ShapeDtypeStruct((G, H), jnp.float32),
        ],
        scratch_shapes=[pltpu.VMEM((G, H), jnp.float32)],
    )(hn_parts, batch3, wm_t, bm_row, wo_t, bo_row)


def kernel(x, concat_feats, srt_concat_batch, end_concat_batch,
           num_concat_feats, batch, W_init, b_init, W_h1, b_h1,
           W_mol, b_mol, W_out, b_out):
    srt1 = srt_concat_batch.astype(jnp.int32)
    end1 = end_concat_batch.astype(jnp.int32)

    def _scatter_layout(idx):
        t = idx.reshape(_SC_NCH, _SCH // 128, 128)
        t = jnp.pad(t, ((0, 0), (0, 8 - _SCH // 128), (0, 0)))
        return t.reshape(_SC_NCH * 8, 128)

    srt_s = _scatter_layout(srt1)
    end_s = _scatter_layout(end1)
    zeros_n = jnp.zeros((N_PAD, H), jnp.float32)
    wh1_t = W_h1.T
    bh1_row = b_h1[None, :]

    h = _tc_h0(concat_feats, W_init.T, b_init[None, :])
    for _ in range(3):
        s_parts = _sc_scatter(h, end_s, zeros_n)
        a, b = _tc_ab(s_parts, lax.slice(h, (0, 0), (N, H)), wh1_t, bh1_row)
        h = _sc_gather_combine(h, a, b, srt1, end1)
    hn_parts = _sc_scatter(h, srt_s, zeros_n)
    atom_embs, out = _tc_final(
        hn_parts, batch.astype(jnp.int32).reshape(N // _BN, 1, _BN),
        W_mol.T, b_mol[None, :], W_out.T, b_out[None, :])
    return (out, atom_embs)


# speedup vs baseline: 4.0213x; 1.6002x over previous
import functools

import jax
import jax.numpy as jnp
from jax import lax
from jax.experimental import pallas as pl
from jax.experimental.pallas import tpu as pltpu
from jax.experimental.pallas import tpu_sc as plsc

N = 10000
N_PAD = 10240
E = 320000
H = 128
D_EDGE = 16
G = 256

NC = 2
NS = 16
NW = NC * NS

_MESH = plsc.VectorSubcoreMesh(
    core_axis_name="c", subcore_axis_name="s", num_cores=NC, num_subcores=NS)


_BE = 2560


def _h0_body(f_ref, w_ref, b_ref, o_ref):
    v = jnp.dot(f_ref[...], w_ref[...], preferred_element_type=jnp.float32)
    v = v + b_ref[...]
    o_ref[...] = jnp.maximum(v, 0.01 * v)


def _tc_h0(feats, w_t, b_row):
    return pl.pallas_call(
        _h0_body,
        grid=(E // _BE,),
        in_specs=[
            pl.BlockSpec((_BE, D_EDGE), lambda i: (i, 0)),
            pl.BlockSpec((D_EDGE, H), lambda i: (0, 0)),
            pl.BlockSpec((1, H), lambda i: (0, 0)),
        ],
        out_specs=pl.BlockSpec((_BE, H), lambda i: (i, 0)),
        out_shape=jax.ShapeDtypeStruct((E, H), jnp.float32),
    )(feats, w_t, b_row)


_SCH = 128
_SC_NCH = E // _SCH
_SC_T = -(-_SC_NCH // NW)
_STRIPE = N_PAD // NS


def _scatter_body(h_hbm, idx_hbm, z_hbm, out_hbm,
                  acc, rows0, rows1, idx0, idx1,
                  sem_ld0, sem_ld1, sem_sc0, sem_sc1):
    rows = [rows0, rows1]
    idxv = [idx0, idx1]
    sem_ld = [sem_ld0, sem_ld1]
    sem_sc = [sem_sc0, sem_sc1]

    cid = lax.axis_index("c")
    sid = lax.axis_index("s")
    wid = sid * NC + cid
    stripe = sid * _STRIPE
    pltpu.sync_copy(z_hbm.at[pl.ds(stripe, _STRIPE)],
                    acc.at[pl.ds(stripe, _STRIPE)])
    plsc.subcore_barrier()

    def valid(t):
        return wid + t * NW < _SC_NCH

    def chunk_of(t):
        return wid + t * NW

    def issue_loads(t, b):
        c = chunk_of(t)
        pltpu.async_copy(idx_hbm.at[pl.ds(c * 8, 8)], idxv[b], sem_ld[b])
        pltpu.async_copy(h_hbm.at[pl.ds(c * _SCH, _SCH)], rows[b], sem_ld[b])

    def wait_loads(b):
        pltpu.make_async_copy(idx_hbm.at[pl.ds(0, 8)], idxv[b],
                              sem_ld[b]).wait()
        pltpu.make_async_copy(h_hbm.at[pl.ds(0, _SCH)], rows[b],
                              sem_ld[b]).wait()

    def wait_scatter(b):
        pltpu.make_async_copy(rows[b], acc.at[pl.ds(0, _SCH)],
                              sem_sc[b]).wait()

    @pl.when(valid(0))
    def _():
        issue_loads(0, 0)

    def step(t, _):
        b0 = lax.rem(t, 2)

        @pl.when((t >= 1) & valid(t - 1))
        def _():
            _wait_scatter_dyn(lax.rem(t + 1, 2))

        @pl.when(valid(t + 1))
        def _():
            _issue_loads_dyn(t + 1, lax.rem(t + 1, 2))

        @pl.when(valid(t))
        def _():
            _wait_loads_dyn(b0)
            _issue_scatter_dyn(t, b0)
        return 0

    def _on_buf(b, fn):
        @pl.when(b == 0)
        def _():
            fn(0)

        @pl.when(b == 1)
        def _():
            fn(1)

    def _issue_loads_dyn(t, b):
        _on_buf(b, lambda bb: issue_loads(t, bb))

    def _wait_loads_dyn(b):
        _on_buf(b, wait_loads)

    def _wait_scatter_dyn(b):
        _on_buf(b, wait_scatter)

    def _issue_scatter_dyn(t, b):
        _on_buf(b, lambda bb: pltpu.async_copy(
            rows[bb], acc.at[idxv[bb].at[0]], sem_sc[bb], add=True))

    lax.fori_loop(0, _SC_T, step, 0)

    @pl.when(valid(_SC_T - 1))
    def _():
        wait_scatter((_SC_T - 1) % 2)

    plsc.subcore_barrier()
    pltpu.sync_copy(acc.at[pl.ds(stripe, _STRIPE)],
                    out_hbm.at[cid, pl.ds(stripe, _STRIPE)])


@functools.partial(
    pl.kernel,
    out_type=jax.ShapeDtypeStruct((NC, N_PAD, H), jnp.float32),
    mesh=_MESH,
    scratch_types=[
        pltpu.VMEM_SHARED((N_PAD, H), jnp.float32),
        pltpu.VMEM((_SCH, H), jnp.float32),
        pltpu.VMEM((_SCH, H), jnp.float32),
        pltpu.VMEM((8, 128), jnp.int32),
        pltpu.VMEM((8, 128), jnp.int32),
        pltpu.SemaphoreType.DMA,
        pltpu.SemaphoreType.DMA,
        pltpu.SemaphoreType.DMA,
        pltpu.SemaphoreType.DMA,
    ],
)
def _sc_scatter(h_hbm, idx_hbm, z_hbm, out_hbm, acc, rows0, rows1,
                idx0, idx1, sem_ld0, sem_ld1, sem_sc0, sem_sc1):
    _scatter_body(h_hbm, idx_hbm, z_hbm, out_hbm, acc, rows0, rows1,
                  idx0, idx1, sem_ld0, sem_ld1, sem_sc0, sem_sc1)


_BN = 2000


def _ab_body(s_ref, h_ref, w_ref, b_ref, a_ref, b_out_ref):
    s = s_ref[0] + s_ref[1]
    a_ref[...] = jnp.dot(s, w_ref[...], preferred_element_type=jnp.float32) \
        + b_ref[...]
    b_out_ref[...] = jnp.dot(h_ref[...], w_ref[...],
                             preferred_element_type=jnp.float32)


def _tc_ab(s_parts, h_n, w_t, b_row):
    return pl.pallas_call(
        _ab_body,
        grid=(N // _BN,),
        in_specs=[
            pl.BlockSpec((NC, _BN, H), lambda i: (0, i, 0)),
            pl.BlockSpec((_BN, H), lambda i: (i, 0)),
            pl.BlockSpec((H, H), lambda i: (0, 0)),
            pl.BlockSpec((1, H), lambda i: (0, 0)),
        ],
        out_specs=[
            pl.BlockSpec((_BN, H), lambda i: (i, 0)),
            pl.BlockSpec((_BN, H), lambda i: (i, 0)),
        ],
        out_shape=[
            jax.ShapeDtypeStruct((N, H), jnp.float32),
            jax.ShapeDtypeStruct((N, H), jnp.float32),
        ],
    )(s_parts, h_n, w_t, b_row)


_GCH = 128
_GC_NCH = E // _GCH
_GC_T = -(-_GC_NCH // NW)


def _gather_body(h_hbm, a_hbm, b_hbm, srt_hbm, end_hbm, out_hbm, *scr):
    h_rows = scr[0:2]
    a_rows = scr[2:4]
    b_rows = scr[4:6]
    sidx = scr[6:9]
    eidx = scr[9:12]
    sem_h = scr[12:14]
    sem_idx = scr[14:17]
    sem_g = scr[17:19]
    sem_st = scr[19:21]

    cid = lax.axis_index("c")
    sid = lax.axis_index("s")
    wid = sid * NC + cid

    def valid(t):
        return wid + t * NW < _GC_NCH

    def chunk_of(t):
        return wid + t * NW

    def issue_idx(t, b):
        c = chunk_of(t)
        pltpu.async_copy(srt_hbm.at[pl.ds(c * _GCH, _GCH)], sidx[b],
                         sem_idx[b])
        pltpu.async_copy(end_hbm.at[pl.ds(c * _GCH, _GCH)], eidx[b],
                         sem_idx[b])

    def wait_idx(b):
        pltpu.make_async_copy(srt_hbm.at[pl.ds(0, _GCH)], sidx[b],
                              sem_idx[b]).wait()
        pltpu.make_async_copy(srt_hbm.at[pl.ds(0, _GCH)], eidx[b],
                              sem_idx[b]).wait()

    def issue_gathers(bi, bg):
        pltpu.async_copy(a_hbm.at[sidx[bi]], a_rows[bg], sem_g[bg])
        pltpu.async_copy(b_hbm.at[eidx[bi]], b_rows[bg], sem_g[bg])

    def wait_gathers(b):
        pltpu.make_async_copy(a_hbm.at[pl.ds(0, _GCH)], a_rows[b],
                              sem_g[b]).wait()
        pltpu.make_async_copy(b_hbm.at[pl.ds(0, _GCH)], b_rows[b],
                              sem_g[b]).wait()

    def issue_h(t, b):
        pltpu.async_copy(h_hbm.at[pl.ds(chunk_of(t) * _GCH, _GCH)],
                         h_rows[b], sem_h[b])

    def wait_h(b):
        pltpu.make_async_copy(h_hbm.at[pl.ds(0, _GCH)], h_rows[b],
                              sem_h[b]).wait()

    def issue_store(t, b):
        pltpu.async_copy(h_rows[b],
                         out_hbm.at[pl.ds(chunk_of(t) * _GCH, _GCH)],
                         sem_st[b])

    def wait_store(t, b):
        pltpu.make_async_copy(h_rows[b],
                              out_hbm.at[pl.ds(0, _GCH)], sem_st[b]).wait()

    def _on2(b, fn):
        @pl.when(b == 0)
        def _():
            fn(0)

        @pl.when(b == 1)
        def _():
            fn(1)

    def _on3(b, fn):
        for bb in range(3):
            @pl.when(b == bb)
            def _(bb=bb):
                fn(bb)

    def combine(b):
        def row(r, _):
            for kk in range(H // 16):
                sl = pl.ds(kk * 16, 16)
                v = h_rows[b][r, sl] + a_rows[b][r, sl] - b_rows[b][r, sl]
                h_rows[b][r, sl] = jnp.maximum(v, 0.01 * v)
            return 0

        lax.fori_loop(0, _GCH, row, 0)

    def _gathers_dyn(t):
        bi = lax.rem(t, 3)
        bg = lax.rem(t, 2)
        for i3 in range(3):
            for i2 in range(2):
                @pl.when((bi == i3) & (bg == i2))
                def _(i3=i3, i2=i2):
                    issue_gathers(i3, i2)

    @pl.when(valid(0))
    def _():
        issue_idx(0, 0)
        wait_idx(0)
        issue_gathers(0, 0)
        issue_h(0, 0)

    @pl.when(valid(1))
    def _():
        issue_idx(1, 1)

    def step(t, _):
        b0 = lax.rem(t, 2)
        b1 = lax.rem(t + 1, 2)

        @pl.when((t >= 1) & valid(t - 1))
        def _():
            _on2(lax.rem(t + 1, 2), lambda bb: wait_store(t - 1, bb))

        @pl.when(valid(t + 1))
        def _():
            _on3(lax.rem(t + 1, 3), wait_idx)
            _gathers_dyn(t + 1)
            _on2(b1, lambda bb: issue_h(t + 1, bb))

        @pl.when(valid(t + 2))
        def _():
            _on3(lax.rem(t + 2, 3), lambda bb: issue_idx(t + 2, bb))

        @pl.when(valid(t))
        def _():
            _on2(b0, wait_gathers)
            _on2(b0, wait_h)
            _on2(b0, combine)
            _on2(b0, lambda bb: issue_store(t, bb))
        return 0

    lax.fori_loop(0, _GC_T, step, 0)

    @pl.when(valid(_GC_T - 1))
    def _():
        wait_store(_GC_T - 1, (_GC_T - 1) % 2)


@functools.partial(
    pl.kernel,
    out_type=jax.ShapeDtypeStruct((E, H), jnp.float32),
    mesh=_MESH,
    scratch_types=(
        [pltpu.VMEM((_GCH, H), jnp.float32)] * 6
        + [pltpu.VMEM((_GCH,), jnp.int32)] * 6
        + [pltpu.SemaphoreType.DMA] * 9
    ),
)
def _sc_gather_combine(h_hbm, a_hbm, b_hbm, srt_hbm, end_hbm, out_hbm, *scr):
    _gather_body(h_hbm, a_hbm, b_hbm, srt_hbm, end_hbm, out_hbm, *scr)


def _final_body(hn_ref, batch_ref, wm_ref, bm_ref, wo_ref, bo_ref,
                ae_ref, out_ref, hg_acc):
    i = pl.program_id(0)
    hn = hn_ref[0] + hn_ref[1]
    ae = jnp.dot(hn, wm_ref[...], preferred_element_type=jnp.float32) \
        + bm_ref[...]
    ae = jnp.maximum(ae, 0.01 * ae)
    ae_ref[...] = ae
    nrm = jnp.sqrt(jnp.sum(ae * ae, axis=1, keepdims=True))
    an = ae / jnp.maximum(nrm, 1e-12)
    bvec = batch_ref[0, 0, :]
    onehot = (bvec[:, None]
              == lax.broadcasted_iota(jnp.int32, (_BN, G), 1)).astype(
                  jnp.float32)
    contrib = lax.dot_general(onehot, an, (((0,), (0,)), ((), ())),
                              preferred_element_type=jnp.float32)

    @pl.when(i == 0)
    def _():
        hg_acc[...] = jnp.zeros_like(hg_acc)

    hg_acc[...] += contrib

    @pl.when(i == N // _BN - 1)
    def _():
        out_ref[...] = jnp.dot(hg_acc[...], wo_ref[...],
                               preferred_element_type=jnp.float32) \
            + bo_ref[...]


def _tc_final(hn_parts, batch3, wm_t, bm_row, wo_t, bo_row):
    return pl.pallas_call(
        _final_body,
        grid=(N // _BN,),
        in_specs=[
            pl.BlockSpec((NC, _BN, H), lambda i: (0, i, 0)),
            pl.BlockSpec((1, 1, _BN), lambda i: (i, 0, 0)),
            pl.BlockSpec((H, H), lambda i: (0, 0)),
            pl.BlockSpec((1, H), lambda i: (0, 0)),
            pl.BlockSpec((H, H), lambda i: (0, 0)),
            pl.BlockSpec((1, H), lambda i: (0, 0)),
        ],
        out_specs=[
            pl.BlockSpec((_BN, H), lambda i: (i, 0)),
            pl.BlockSpec((G, H), lambda i: (0, 0)),
        ],
        out_shape=[
            jax.ShapeDtypeStruct((N, H), jnp.float32),
            jax.ShapeDtypeStruct((G, H), jnp.float32),
        ],
        scratch_shapes=[pltpu.VMEM((G, H), jnp.float32)],
    )(hn_parts, batch3, wm_t, bm_row, wo_t, bo_row)


def kernel(x, concat_feats, srt_concat_batch, end_concat_batch,
           num_concat_feats, batch, W_init, b_init, W_h1, b_h1,
           W_mol, b_mol, W_out, b_out):
    srt1 = srt_concat_batch.astype(jnp.int32)
    end1 = end_concat_batch.astype(jnp.int32)

    def _scatter_layout(idx):
        t = idx.reshape(_SC_NCH, _SCH // 128, 128)
        t = jnp.pad(t, ((0, 0), (0, 8 - _SCH // 128), (0, 0)))
        return t.reshape(_SC_NCH * 8, 128)

    srt_s = _scatter_layout(srt1)
    end_s = _scatter_layout(end1)
    zeros_n = jnp.zeros((N_PAD, H), jnp.float32)
    wh1_t = W_h1.T
    bh1_row = b_h1[None, :]

    h = _tc_h0(concat_feats, W_init.T, b_init[None, :])
    for _ in range(3):
        s_parts = _sc_scatter(h, end_s, zeros_n)
        a, b = _tc_ab(s_parts, lax.slice(h, (0, 0), (N, H)), wh1_t, bh1_row)
        h = _sc_gather_combine(h, a, b, srt1, end1)
    hn_parts = _sc_scatter(h, srt_s, zeros_n)
    atom_embs, out = _tc_final(
        hn_parts, batch.astype(jnp.int32).reshape(N // _BN, 1, _BN),
        W_mol.T, b_mol[None, :], W_out.T, b_out[None, :])
    return (out, atom_embs)


# speedup vs baseline: 4.7290x; 1.1760x over previous
import functools

import jax
import jax.numpy as jnp
from jax import lax
from jax.experimental import pallas as pl
from jax.experimental.pallas import tpu as pltpu
from jax.experimental.pallas import tpu_sc as plsc

N = 10000
N_PAD = 10240
E = 320000
H = 128
D_EDGE = 16
G = 256

NC = 2
NS = 16
NW = NC * NS

_MESH = plsc.VectorSubcoreMesh(
    core_axis_name="c", subcore_axis_name="s", num_cores=NC, num_subcores=NS)


_BE = 2560


def _h0_body(f_ref, w_ref, b_ref, o_ref):
    v = jnp.dot(f_ref[...], w_ref[...], preferred_element_type=jnp.float32)
    v = v + b_ref[...]
    o_ref[...] = jnp.maximum(v, 0.01 * v)


def _tc_h0(feats, w_t, b_row):
    return pl.pallas_call(
        _h0_body,
        grid=(E // _BE,),
        in_specs=[
            pl.BlockSpec((_BE, D_EDGE), lambda i: (i, 0)),
            pl.BlockSpec((D_EDGE, H), lambda i: (0, 0)),
            pl.BlockSpec((1, H), lambda i: (0, 0)),
        ],
        out_specs=pl.BlockSpec((_BE, H), lambda i: (i, 0)),
        out_shape=jax.ShapeDtypeStruct((E, H), jnp.float32),
    )(feats, w_t, b_row)


_SCH = 128
_SC_NCH = E // _SCH
_SC_T = -(-_SC_NCH // NW)
_STRIPE = N_PAD // NS


def _scatter_body(h_hbm, idx_hbm, z_hbm, out_hbm,
                  acc, rows0, rows1, idx0, idx1,
                  sem_ld0, sem_ld1, sem_sc0, sem_sc1):
    rows = [rows0, rows1]
    idxv = [idx0, idx1]
    sem_ld = [sem_ld0, sem_ld1]
    sem_sc = [sem_sc0, sem_sc1]

    cid = lax.axis_index("c")
    sid = lax.axis_index("s")
    wid = sid * NC + cid
    stripe = sid * _STRIPE
    pltpu.sync_copy(z_hbm.at[pl.ds(stripe, _STRIPE)],
                    acc.at[pl.ds(stripe, _STRIPE)])
    plsc.subcore_barrier()

    def valid(t):
        return wid + t * NW < _SC_NCH

    def chunk_of(t):
        return wid + t * NW

    def issue_loads(t, b):
        c = chunk_of(t)
        pltpu.async_copy(idx_hbm.at[pl.ds(c * 8, 8)], idxv[b], sem_ld[b])
        pltpu.async_copy(h_hbm.at[pl.ds(c * _SCH, _SCH)], rows[b], sem_ld[b])

    def wait_loads(b):
        pltpu.make_async_copy(idx_hbm.at[pl.ds(0, 8)], idxv[b],
                              sem_ld[b]).wait()
        pltpu.make_async_copy(h_hbm.at[pl.ds(0, _SCH)], rows[b],
                              sem_ld[b]).wait()

    def wait_scatter(b):
        pltpu.make_async_copy(rows[b], acc.at[pl.ds(0, _SCH)],
                              sem_sc[b]).wait()

    @pl.when(valid(0))
    def _():
        issue_loads(0, 0)

    def step(t, _):
        b0 = lax.rem(t, 2)

        @pl.when((t >= 1) & valid(t - 1))
        def _():
            _wait_scatter_dyn(lax.rem(t + 1, 2))

        @pl.when(valid(t + 1))
        def _():
            _issue_loads_dyn(t + 1, lax.rem(t + 1, 2))

        @pl.when(valid(t))
        def _():
            _wait_loads_dyn(b0)
            _issue_scatter_dyn(t, b0)
        return 0

    def _on_buf(b, fn):
        @pl.when(b == 0)
        def _():
            fn(0)

        @pl.when(b == 1)
        def _():
            fn(1)

    def _issue_loads_dyn(t, b):
        _on_buf(b, lambda bb: issue_loads(t, bb))

    def _wait_loads_dyn(b):
        _on_buf(b, wait_loads)

    def _wait_scatter_dyn(b):
        _on_buf(b, wait_scatter)

    def _issue_scatter_dyn(t, b):
        _on_buf(b, lambda bb: pltpu.async_copy(
            rows[bb], acc.at[idxv[bb].at[0]], sem_sc[bb], add=True))

    lax.fori_loop(0, _SC_T, step, 0)

    @pl.when(valid(_SC_T - 1))
    def _():
        wait_scatter((_SC_T - 1) % 2)

    plsc.subcore_barrier()
    pltpu.sync_copy(acc.at[pl.ds(stripe, _STRIPE)],
                    out_hbm.at[cid, pl.ds(stripe, _STRIPE)])


@functools.partial(
    pl.kernel,
    out_type=jax.ShapeDtypeStruct((NC, N_PAD, H), jnp.float32),
    mesh=_MESH,
    scratch_types=[
        pltpu.VMEM_SHARED((N_PAD, H), jnp.float32),
        pltpu.VMEM((_SCH, H), jnp.float32),
        pltpu.VMEM((_SCH, H), jnp.float32),
        pltpu.VMEM((8, 128), jnp.int32),
        pltpu.VMEM((8, 128), jnp.int32),
        pltpu.SemaphoreType.DMA,
        pltpu.SemaphoreType.DMA,
        pltpu.SemaphoreType.DMA,
        pltpu.SemaphoreType.DMA,
    ],
)
def _sc_scatter(h_hbm, idx_hbm, z_hbm, out_hbm, acc, rows0, rows1,
                idx0, idx1, sem_ld0, sem_ld1, sem_sc0, sem_sc1):
    _scatter_body(h_hbm, idx_hbm, z_hbm, out_hbm, acc, rows0, rows1,
                  idx0, idx1, sem_ld0, sem_ld1, sem_sc0, sem_sc1)


_BN = 2000


def _ab_body(s_ref, h_ref, w_ref, b_ref, a_ref, b_out_ref):
    s = s_ref[0] + s_ref[1]
    a_ref[...] = jnp.dot(s, w_ref[...], preferred_element_type=jnp.float32) \
        + b_ref[...]
    b_out_ref[...] = jnp.dot(h_ref[...], w_ref[...],
                             preferred_element_type=jnp.float32)


def _tc_ab(s_parts, h_n, w_t, b_row):
    return pl.pallas_call(
        _ab_body,
        grid=(N // _BN,),
        in_specs=[
            pl.BlockSpec((NC, _BN, H), lambda i: (0, i, 0)),
            pl.BlockSpec((_BN, H), lambda i: (i, 0)),
            pl.BlockSpec((H, H), lambda i: (0, 0)),
            pl.BlockSpec((1, H), lambda i: (0, 0)),
        ],
        out_specs=[
            pl.BlockSpec((_BN, H), lambda i: (i, 0)),
            pl.BlockSpec((_BN, H), lambda i: (i, 0)),
        ],
        out_shape=[
            jax.ShapeDtypeStruct((N, H), jnp.float32),
            jax.ShapeDtypeStruct((N, H), jnp.float32),
        ],
    )(s_parts, h_n, w_t, b_row)


_GCH = 128
_GC_NCH = E // _GCH
_GC_T = -(-_GC_NCH // NW)


def _gather_body(h_hbm, a_hbm, b_hbm, srt_hbm, end_hbm, out_hbm, *scr):
    h_rows = scr[0:2]
    a_rows = scr[2:4]
    b_rows = scr[4:6]
    sidx = scr[6:9]
    eidx = scr[9:12]
    sem_h = scr[12:14]
    sem_idx = scr[14:17]
    sem_g = scr[17:19]
    sem_st = scr[19:21]

    cid = lax.axis_index("c")
    sid = lax.axis_index("s")
    wid = sid * NC + cid

    def valid(t):
        return wid + t * NW < _GC_NCH

    def chunk_of(t):
        return wid + t * NW

    def issue_idx(t, b):
        c = chunk_of(t)
        pltpu.async_copy(srt_hbm.at[pl.ds(c * _GCH, _GCH)], sidx[b],
                         sem_idx[b])
        pltpu.async_copy(end_hbm.at[pl.ds(c * _GCH, _GCH)], eidx[b],
                         sem_idx[b])

    def wait_idx(b):
        pltpu.make_async_copy(srt_hbm.at[pl.ds(0, _GCH)], sidx[b],
                              sem_idx[b]).wait()
        pltpu.make_async_copy(srt_hbm.at[pl.ds(0, _GCH)], eidx[b],
                              sem_idx[b]).wait()

    def issue_gathers(bi, bg):
        pltpu.async_copy(a_hbm.at[sidx[bi]], a_rows[bg], sem_g[bg])
        pltpu.async_copy(b_hbm.at[eidx[bi]], b_rows[bg], sem_g[bg])

    def wait_gathers(b):
        pltpu.make_async_copy(a_hbm.at[pl.ds(0, _GCH)], a_rows[b],
                              sem_g[b]).wait()
        pltpu.make_async_copy(b_hbm.at[pl.ds(0, _GCH)], b_rows[b],
                              sem_g[b]).wait()

    def issue_h(t, b):
        pltpu.async_copy(h_hbm.at[pl.ds(chunk_of(t) * _GCH, _GCH)],
                         h_rows[b], sem_h[b])

    def wait_h(b):
        pltpu.make_async_copy(h_hbm.at[pl.ds(0, _GCH)], h_rows[b],
                              sem_h[b]).wait()

    def issue_store(t, b):
        pltpu.async_copy(h_rows[b],
                         out_hbm.at[pl.ds(chunk_of(t) * _GCH, _GCH)],
                         sem_st[b])

    def wait_store(t, b):
        pltpu.make_async_copy(h_rows[b],
                              out_hbm.at[pl.ds(0, _GCH)], sem_st[b]).wait()

    def _on2(b, fn):
        @pl.when(b == 0)
        def _():
            fn(0)

        @pl.when(b == 1)
        def _():
            fn(1)

    def _on3(b, fn):
        for bb in range(3):
            @pl.when(b == bb)
            def _(bb=bb):
                fn(bb)

    def combine(b):
        def row(r, _):
            for kk in range(H // 16):
                sl = pl.ds(kk * 16, 16)
                v = h_rows[b][r, sl] + a_rows[b][r, sl] - b_rows[b][r, sl]
                h_rows[b][r, sl] = jnp.maximum(v, 0.01 * v)
            return 0

        lax.fori_loop(0, _GCH, row, 0)

    def _gathers_dyn(t):
        bi = lax.rem(t, 3)
        bg = lax.rem(t, 2)
        for i3 in range(3):
            for i2 in range(2):
                @pl.when((bi == i3) & (bg == i2))
                def _(i3=i3, i2=i2):
                    issue_gathers(i3, i2)

    @pl.when(valid(0))
    def _():
        issue_idx(0, 0)
        wait_idx(0)
        issue_gathers(0, 0)
        issue_h(0, 0)

    @pl.when(valid(1))
    def _():
        issue_idx(1, 1)

    def step(t, _):
        b0 = lax.rem(t, 2)
        b1 = lax.rem(t + 1, 2)

        @pl.when((t >= 1) & valid(t - 1))
        def _():
            _on2(lax.rem(t + 1, 2), lambda bb: wait_store(t - 1, bb))

        @pl.when(valid(t + 1))
        def _():
            _on3(lax.rem(t + 1, 3), wait_idx)
            _gathers_dyn(t + 1)
            _on2(b1, lambda bb: issue_h(t + 1, bb))

        @pl.when(valid(t + 2))
        def _():
            _on3(lax.rem(t + 2, 3), lambda bb: issue_idx(t + 2, bb))

        @pl.when(valid(t))
        def _():
            _on2(b0, wait_gathers)
            _on2(b0, wait_h)
            _on2(b0, combine)
            _on2(b0, lambda bb: issue_store(t, bb))
        return 0

    lax.fori_loop(0, _GC_T, step, 0)

    @pl.when(valid(_GC_T - 1))
    def _():
        wait_store(_GC_T - 1, (_GC_T - 1) % 2)


@functools.partial(
    pl.kernel,
    out_type=jax.ShapeDtypeStruct((E, H), jnp.float32),
    mesh=_MESH,
    scratch_types=(
        [pltpu.VMEM((_GCH, H), jnp.float32)] * 6
        + [pltpu.VMEM((_GCH,), jnp.int32)] * 6
        + [pltpu.SemaphoreType.DMA] * 9
    ),
)
def _sc_gather_combine(h_hbm, a_hbm, b_hbm, srt_hbm, end_hbm, out_hbm, *scr):
    _gather_body(h_hbm, a_hbm, b_hbm, srt_hbm, end_hbm, out_hbm, *scr)


_FCH = 64
_F_NCH = E // _FCH
_F_T = -(-_F_NCH // NW)


def _make_fused(write_out, scat_from_srt):
    def body(h_hbm, a_hbm, b_hbm, srt_hbm, end_hbm, z_hbm, *rest):
        if write_out:
            s_out, h_out = rest[0], rest[1]
            scr = rest[2:]
        else:
            s_out = rest[0]
            h_out = None
            scr = rest[1:]
        acc = scr[0]
        h_rows = scr[1:3]
        a_rows = scr[3:5]
        b_rows = scr[5:7]
        sidx = scr[7:10]
        eidx = scr[10:13]
        sem_h = scr[13:15]
        sem_idx = scr[15:18]
        sem_g = scr[18:20]
        sem_st = scr[20:22]
        sem_sc = scr[22:24]
        scat = sidx if scat_from_srt else eidx

        cid = lax.axis_index("c")
        sid = lax.axis_index("s")
        wid = sid * NC + cid

        @pl.when(sid == 0)
        def _():
            pltpu.sync_copy(z_hbm, acc)

        plsc.subcore_barrier()

        def valid(t):
            return wid + t * NW < _F_NCH

        def chunk_of(t):
            return wid + t * NW

        def issue_idx(t, b):
            c = chunk_of(t)
            pltpu.async_copy(srt_hbm.at[pl.ds(c * _FCH, _FCH)], sidx[b],
                             sem_idx[b])
            pltpu.async_copy(end_hbm.at[pl.ds(c * _FCH, _FCH)], eidx[b],
                             sem_idx[b])

        def wait_idx(b):
            pltpu.make_async_copy(srt_hbm.at[pl.ds(0, _FCH)], sidx[b],
                                  sem_idx[b]).wait()
            pltpu.make_async_copy(srt_hbm.at[pl.ds(0, _FCH)], eidx[b],
                                  sem_idx[b]).wait()

        def issue_gathers(bi, bg):
            pltpu.async_copy(a_hbm.at[sidx[bi]], a_rows[bg], sem_g[bg])
            pltpu.async_copy(b_hbm.at[eidx[bi]], b_rows[bg], sem_g[bg])

        def wait_gathers(b):
            pltpu.make_async_copy(a_hbm.at[pl.ds(0, _FCH)], a_rows[b],
                                  sem_g[b]).wait()
            pltpu.make_async_copy(b_hbm.at[pl.ds(0, _FCH)], b_rows[b],
                                  sem_g[b]).wait()

        def issue_h(t, b):
            pltpu.async_copy(h_hbm.at[pl.ds(chunk_of(t) * _FCH, _FCH)],
                             h_rows[b], sem_h[b])

        def wait_h(b):
            pltpu.make_async_copy(h_hbm.at[pl.ds(0, _FCH)], h_rows[b],
                                  sem_h[b]).wait()

        def issue_store(t, b):
            pltpu.async_copy(h_rows[b],
                             h_out.at[pl.ds(chunk_of(t) * _FCH, _FCH)],
                             sem_st[b])

        def wait_store(b):
            pltpu.make_async_copy(h_rows[b], h_out.at[pl.ds(0, _FCH)],
                                  sem_st[b]).wait()

        def issue_sc(bi, bg):
            for g in range(_FCH // 16):
                vec = scat[bi][pl.ds(g * 16, 16)]
                pltpu.async_copy(h_rows[bg].at[pl.ds(g * 16, 16)],
                                 acc.at[vec], sem_sc[bg], add=True)

        def wait_sc(b):
            for g in range(_FCH // 16):
                pltpu.make_async_copy(h_rows[b].at[pl.ds(0, 16)],
                                      acc.at[pl.ds(0, 16)], sem_sc[b]).wait()

        def _on2(b, fn):
            for bb in range(2):
                @pl.when(b == bb)
                def _(bb=bb):
                    fn(bb)

        def _on3(b, fn):
            for bb in range(3):
                @pl.when(b == bb)
                def _(bb=bb):
                    fn(bb)

        def _on32(bi, bg, fn):
            for i3 in range(3):
                for i2 in range(2):
                    @pl.when((bi == i3) & (bg == i2))
                    def _(i3=i3, i2=i2):
                        fn(i3, i2)

        def combine(b):
            def row(r, _):
                for kk in range(H // 16):
                    sl = pl.ds(kk * 16, 16)
                    v = h_rows[b][r, sl] + a_rows[b][r, sl] \
                        - b_rows[b][r, sl]
                    h_rows[b][r, sl] = jnp.maximum(v, 0.01 * v)
                return 0

            lax.fori_loop(0, _FCH, row, 0)

        @pl.when(valid(0))
        def _():
            issue_idx(0, 0)
            wait_idx(0)
            issue_gathers(0, 0)
            issue_h(0, 0)

        @pl.when(valid(1))
        def _():
            issue_idx(1, 1)

        def step(t, _):
            b0 = lax.rem(t, 2)
            b1 = lax.rem(t + 1, 2)

            @pl.when((t >= 1) & valid(t - 1))
            def _():
                if write_out:
                    _on2(lax.rem(t + 1, 2), wait_store)
                _on2(lax.rem(t + 1, 2), wait_sc)

            @pl.when(valid(t + 1))
            def _():
                _on3(lax.rem(t + 1, 3), wait_idx)
                _on32(lax.rem(t + 1, 3), b1, issue_gathers)
                _on2(b1, lambda bb: issue_h(t + 1, bb))

            @pl.when(valid(t + 2))
            def _():
                _on3(lax.rem(t + 2, 3), lambda bb: issue_idx(t + 2, bb))

            @pl.when(valid(t))
            def _():
                _on2(b0, wait_gathers)
                _on2(b0, wait_h)
                _on2(b0, combine)
                if write_out:
                    _on2(b0, lambda bb: issue_store(t, bb))
                _on32(lax.rem(t, 3), b0, issue_sc)
            return 0

        lax.fori_loop(0, _F_T, step, 0)

        @pl.when(valid(_F_T - 1))
        def _():
            if write_out:
                wait_store((_F_T - 1) % 2)
            wait_sc((_F_T - 1) % 2)

        plsc.subcore_barrier()

        @pl.when(sid == 0)
        def _():
            pltpu.sync_copy(acc, s_out.at[cid])

    if write_out:
        outs = [jax.ShapeDtypeStruct((NC, N, H), jnp.float32),
                jax.ShapeDtypeStruct((E, H), jnp.float32)]
    else:
        outs = jax.ShapeDtypeStruct((NC, N, H), jnp.float32)
    return pl.kernel(
        body,
        out_type=outs,
        mesh=_MESH,
        scratch_types=(
            [pltpu.VMEM_SHARED((N, H), jnp.float32)]
            + [pltpu.VMEM((_FCH, H), jnp.float32)] * 6
            + [pltpu.VMEM((_FCH,), jnp.int32)] * 6
            + [pltpu.SemaphoreType.DMA] * 11
        ),
    )


_FUSED_MID = _make_fused(True, False)
_FUSED_LAST = _make_fused(False, True)


def _final_body(hn_ref, batch_ref, wm_ref, bm_ref, wo_ref, bo_ref,
                ae_ref, out_ref, hg_acc):
    i = pl.program_id(0)
    hn = hn_ref[0] + hn_ref[1]
    ae = jnp.dot(hn, wm_ref[...], preferred_element_type=jnp.float32) \
        + bm_ref[...]
    ae = jnp.maximum(ae, 0.01 * ae)
    ae_ref[...] = ae
    nrm = jnp.sqrt(jnp.sum(ae * ae, axis=1, keepdims=True))
    an = ae / jnp.maximum(nrm, 1e-12)
    bvec = batch_ref[0, 0, :]
    onehot = (bvec[:, None]
              == lax.broadcasted_iota(jnp.int32, (_BN, G), 1)).astype(
                  jnp.float32)
    contrib = lax.dot_general(onehot, an, (((0,), (0,)), ((), ())),
                              preferred_element_type=jnp.float32)

    @pl.when(i == 0)
    def _():
        hg_acc[...] = jnp.zeros_like(hg_acc)

    hg_acc[...] += contrib

    @pl.when(i == N // _BN - 1)
    def _():
        out_ref[...] = jnp.dot(hg_acc[...], wo_ref[...],
                               preferred_element_type=jnp.float32) \
            + bo_ref[...]


def _tc_final(hn_parts, batch3, wm_t, bm_row, wo_t, bo_row):
    return pl.pallas_call(
        _final_body,
        grid=(N // _BN,),
        in_specs=[
            pl.BlockSpec((NC, _BN, H), lambda i: (0, i, 0)),
            pl.BlockSpec((1, 1, _BN), lambda i: (i, 0, 0)),
            pl.BlockSpec((H, H), lambda i: (0, 0)),
            pl.BlockSpec((1, H), lambda i: (0, 0)),
            pl.BlockSpec((H, H), lambda i: (0, 0)),
            pl.BlockSpec((1, H), lambda i: (0, 0)),
        ],
        out_specs=[
            pl.BlockSpec((_BN, H), lambda i: (i, 0)),
            pl.BlockSpec((G, H), lambda i: (0, 0)),
        ],
        out_shape=[
            jax.ShapeDtypeStruct((N, H), jnp.float32),
            jax.ShapeDtypeStruct((G, H), jnp.float32),
        ],
        scratch_shapes=[pltpu.VMEM((G, H), jnp.float32)],
    )(hn_parts, batch3, wm_t, bm_row, wo_t, bo_row)


def kernel(x, concat_feats, srt_concat_batch, end_concat_batch,
           num_concat_feats, batch, W_init, b_init, W_h1, b_h1,
           W_mol, b_mol, W_out, b_out):
    srt1 = srt_concat_batch.astype(jnp.int32)
    end1 = end_concat_batch.astype(jnp.int32)

    def _scatter_layout(idx):
        t = idx.reshape(_SC_NCH, _SCH // 128, 128)
        t = jnp.pad(t, ((0, 0), (0, 8 - _SCH // 128), (0, 0)))
        return t.reshape(_SC_NCH * 8, 128)

    end_s = _scatter_layout(end1)
    zeros_n = jnp.zeros((N_PAD, H), jnp.float32)
    wh1_t = W_h1.T
    bh1_row = b_h1[None, :]

    zeros_acc = jnp.zeros((N, H), jnp.float32)

    h = _tc_h0(concat_feats, W_init.T, b_init[None, :])
    s_parts = _sc_scatter(h, end_s, zeros_n)
    for layer in range(3):
        a, b = _tc_ab(s_parts, lax.slice(h, (0, 0), (N, H)), wh1_t, bh1_row)
        if layer < 2:
            s_parts, h = _FUSED_MID(h, a, b, srt1, end1, zeros_acc)
        else:
            hn_parts = _FUSED_LAST(h, a, b, srt1, end1, zeros_acc)
    atom_embs, out = _tc_final(
        hn_parts, batch.astype(jnp.int32).reshape(N // _BN, 1, _BN),
        W_mol.T, b_mol[None, :], W_out.T, b_out[None, :])
    return (out, atom_embs)


# speedup vs baseline: 4.7994x; 1.0149x over previous
import functools

import jax
import jax.numpy as jnp
from jax import lax
from jax.experimental import pallas as pl
from jax.experimental.pallas import tpu as pltpu
from jax.experimental.pallas import tpu_sc as plsc

N = 10000
N_PAD = 10240
E = 320000
H = 128
D_EDGE = 16
G = 256

NC = 2
NS = 16
NW = NC * NS

_MESH = plsc.VectorSubcoreMesh(
    core_axis_name="c", subcore_axis_name="s", num_cores=NC, num_subcores=NS)


_BE = 2560


def _h0_body(f_ref, w_ref, b_ref, o_ref):
    v = jnp.dot(f_ref[...], w_ref[...], preferred_element_type=jnp.float32)
    v = v + b_ref[...]
    o_ref[...] = jnp.maximum(v, 0.01 * v)


def _tc_h0(feats, w_t, b_row):
    return pl.pallas_call(
        _h0_body,
        grid=(E // _BE,),
        in_specs=[
            pl.BlockSpec((_BE, D_EDGE), lambda i: (i, 0)),
            pl.BlockSpec((D_EDGE, H), lambda i: (0, 0)),
            pl.BlockSpec((1, H), lambda i: (0, 0)),
        ],
        out_specs=pl.BlockSpec((_BE, H), lambda i: (i, 0)),
        out_shape=jax.ShapeDtypeStruct((E, H), jnp.float32),
    )(feats, w_t, b_row)


_SCH = 128
_SC_NCH = E // _SCH
_SC_T = -(-_SC_NCH // NW)
_STRIPE = N_PAD // NS


def _scatter_body(h_hbm, idx_hbm, z_hbm, out_hbm,
                  acc, rows0, rows1, idx0, idx1,
                  sem_ld0, sem_ld1, sem_sc0, sem_sc1):
    rows = [rows0, rows1]
    idxv = [idx0, idx1]
    sem_ld = [sem_ld0, sem_ld1]
    sem_sc = [sem_sc0, sem_sc1]

    cid = lax.axis_index("c")
    sid = lax.axis_index("s")
    wid = sid * NC + cid
    stripe = sid * _STRIPE
    pltpu.sync_copy(z_hbm.at[pl.ds(stripe, _STRIPE)],
                    acc.at[pl.ds(stripe, _STRIPE)])
    plsc.subcore_barrier()

    def valid(t):
        return wid + t * NW < _SC_NCH

    def chunk_of(t):
        return wid + t * NW

    def issue_loads(t, b):
        c = chunk_of(t)
        pltpu.async_copy(idx_hbm.at[pl.ds(c * _SCH, _SCH)], idxv[b],
                         sem_ld[b])
        pltpu.async_copy(h_hbm.at[pl.ds(c * _SCH, _SCH)], rows[b], sem_ld[b])

    def wait_loads(b):
        pltpu.make_async_copy(idx_hbm.at[pl.ds(0, _SCH)], idxv[b],
                              sem_ld[b]).wait()
        pltpu.make_async_copy(h_hbm.at[pl.ds(0, _SCH)], rows[b],
                              sem_ld[b]).wait()

    def wait_scatter(b):
        for g in range(_SCH // 16):
            pltpu.make_async_copy(rows[b].at[pl.ds(0, 16)],
                                  acc.at[pl.ds(0, 16)], sem_sc[b]).wait()

    @pl.when(valid(0))
    def _():
        issue_loads(0, 0)

    def step(t, _):
        b0 = lax.rem(t, 2)

        @pl.when((t >= 1) & valid(t - 1))
        def _():
            _wait_scatter_dyn(lax.rem(t + 1, 2))

        @pl.when(valid(t + 1))
        def _():
            _issue_loads_dyn(t + 1, lax.rem(t + 1, 2))

        @pl.when(valid(t))
        def _():
            _wait_loads_dyn(b0)
            _issue_scatter_dyn(t, b0)
        return 0

    def _on_buf(b, fn):
        @pl.when(b == 0)
        def _():
            fn(0)

        @pl.when(b == 1)
        def _():
            fn(1)

    def _issue_loads_dyn(t, b):
        _on_buf(b, lambda bb: issue_loads(t, bb))

    def _wait_loads_dyn(b):
        _on_buf(b, wait_loads)

    def _wait_scatter_dyn(b):
        _on_buf(b, wait_scatter)

    def _issue_scatter_dyn(t, b):
        def go(bb):
            for g in range(_SCH // 16):
                vec = idxv[bb][pl.ds(g * 16, 16)]
                pltpu.async_copy(rows[bb].at[pl.ds(g * 16, 16)],
                                 acc.at[vec], sem_sc[bb], add=True)

        _on_buf(b, go)

    lax.fori_loop(0, _SC_T, step, 0)

    @pl.when(valid(_SC_T - 1))
    def _():
        wait_scatter((_SC_T - 1) % 2)

    plsc.subcore_barrier()
    pltpu.sync_copy(acc.at[pl.ds(stripe, _STRIPE)],
                    out_hbm.at[cid, pl.ds(stripe, _STRIPE)])


@functools.partial(
    pl.kernel,
    out_type=jax.ShapeDtypeStruct((NC, N_PAD, H), jnp.float32),
    mesh=_MESH,
    scratch_types=[
        pltpu.VMEM_SHARED((N_PAD, H), jnp.float32),
        pltpu.VMEM((_SCH, H), jnp.float32),
        pltpu.VMEM((_SCH, H), jnp.float32),
        pltpu.VMEM((_SCH,), jnp.int32),
        pltpu.VMEM((_SCH,), jnp.int32),
        pltpu.SemaphoreType.DMA,
        pltpu.SemaphoreType.DMA,
        pltpu.SemaphoreType.DMA,
        pltpu.SemaphoreType.DMA,
    ],
)
def _sc_scatter(h_hbm, idx_hbm, z_hbm, out_hbm, acc, rows0, rows1,
                idx0, idx1, sem_ld0, sem_ld1, sem_sc0, sem_sc1):
    _scatter_body(h_hbm, idx_hbm, z_hbm, out_hbm, acc, rows0, rows1,
                  idx0, idx1, sem_ld0, sem_ld1, sem_sc0, sem_sc1)


_BN = 2000


def _ab_body(s_ref, h_ref, w_ref, b_ref, a_ref, b_out_ref):
    s = s_ref[0] + s_ref[1]
    a_ref[...] = jnp.dot(s, w_ref[...], preferred_element_type=jnp.float32) \
        + b_ref[...]
    b_out_ref[...] = jnp.dot(h_ref[...], w_ref[...],
                             preferred_element_type=jnp.float32)


def _tc_ab(s_parts, h_n, w_t, b_row):
    return pl.pallas_call(
        _ab_body,
        grid=(N // _BN,),
        in_specs=[
            pl.BlockSpec((NC, _BN, H), lambda i: (0, i, 0)),
            pl.BlockSpec((_BN, H), lambda i: (i, 0)),
            pl.BlockSpec((H, H), lambda i: (0, 0)),
            pl.BlockSpec((1, H), lambda i: (0, 0)),
        ],
        out_specs=[
            pl.BlockSpec((_BN, H), lambda i: (i, 0)),
            pl.BlockSpec((_BN, H), lambda i: (i, 0)),
        ],
        out_shape=[
            jax.ShapeDtypeStruct((N, H), jnp.float32),
            jax.ShapeDtypeStruct((N, H), jnp.float32),
        ],
    )(s_parts, h_n, w_t, b_row)


_GCH = 128
_GC_NCH = E // _GCH
_GC_T = -(-_GC_NCH // NW)


def _gather_body(h_hbm, a_hbm, b_hbm, srt_hbm, end_hbm, out_hbm, *scr):
    h_rows = scr[0:2]
    a_rows = scr[2:4]
    b_rows = scr[4:6]
    sidx = scr[6:9]
    eidx = scr[9:12]
    sem_h = scr[12:14]
    sem_idx = scr[14:17]
    sem_g = scr[17:19]
    sem_st = scr[19:21]

    cid = lax.axis_index("c")
    sid = lax.axis_index("s")
    wid = sid * NC + cid

    def valid(t):
        return wid + t * NW < _GC_NCH

    def chunk_of(t):
        return wid + t * NW

    def issue_idx(t, b):
        c = chunk_of(t)
        pltpu.async_copy(srt_hbm.at[pl.ds(c * _GCH, _GCH)], sidx[b],
                         sem_idx[b])
        pltpu.async_copy(end_hbm.at[pl.ds(c * _GCH, _GCH)], eidx[b],
                         sem_idx[b])

    def wait_idx(b):
        pltpu.make_async_copy(srt_hbm.at[pl.ds(0, _GCH)], sidx[b],
                              sem_idx[b]).wait()
        pltpu.make_async_copy(srt_hbm.at[pl.ds(0, _GCH)], eidx[b],
                              sem_idx[b]).wait()

    def issue_gathers(bi, bg):
        pltpu.async_copy(a_hbm.at[sidx[bi]], a_rows[bg], sem_g[bg])
        pltpu.async_copy(b_hbm.at[eidx[bi]], b_rows[bg], sem_g[bg])

    def wait_gathers(b):
        pltpu.make_async_copy(a_hbm.at[pl.ds(0, _GCH)], a_rows[b],
                              sem_g[b]).wait()
        pltpu.make_async_copy(b_hbm.at[pl.ds(0, _GCH)], b_rows[b],
                              sem_g[b]).wait()

    def issue_h(t, b):
        pltpu.async_copy(h_hbm.at[pl.ds(chunk_of(t) * _GCH, _GCH)],
                         h_rows[b], sem_h[b])

    def wait_h(b):
        pltpu.make_async_copy(h_hbm.at[pl.ds(0, _GCH)], h_rows[b],
                              sem_h[b]).wait()

    def issue_store(t, b):
        pltpu.async_copy(h_rows[b],
                         out_hbm.at[pl.ds(chunk_of(t) * _GCH, _GCH)],
                         sem_st[b])

    def wait_store(t, b):
        pltpu.make_async_copy(h_rows[b],
                              out_hbm.at[pl.ds(0, _GCH)], sem_st[b]).wait()

    def _on2(b, fn):
        @pl.when(b == 0)
        def _():
            fn(0)

        @pl.when(b == 1)
        def _():
            fn(1)

    def _on3(b, fn):
        for bb in range(3):
            @pl.when(b == bb)
            def _(bb=bb):
                fn(bb)

    def combine(b):
        def row(r, _):
            for kk in range(H // 16):
                sl = pl.ds(kk * 16, 16)
                v = h_rows[b][r, sl] + a_rows[b][r, sl] - b_rows[b][r, sl]
                h_rows[b][r, sl] = jnp.maximum(v, 0.01 * v)
            return 0

        lax.fori_loop(0, _GCH, row, 0)

    def _gathers_dyn(t):
        bi = lax.rem(t, 3)
        bg = lax.rem(t, 2)
        for i3 in range(3):
            for i2 in range(2):
                @pl.when((bi == i3) & (bg == i2))
                def _(i3=i3, i2=i2):
                    issue_gathers(i3, i2)

    @pl.when(valid(0))
    def _():
        issue_idx(0, 0)
        wait_idx(0)
        issue_gathers(0, 0)
        issue_h(0, 0)

    @pl.when(valid(1))
    def _():
        issue_idx(1, 1)

    def step(t, _):
        b0 = lax.rem(t, 2)
        b1 = lax.rem(t + 1, 2)

        @pl.when((t >= 1) & valid(t - 1))
        def _():
            _on2(lax.rem(t + 1, 2), lambda bb: wait_store(t - 1, bb))

        @pl.when(valid(t + 1))
        def _():
            _on3(lax.rem(t + 1, 3), wait_idx)
            _gathers_dyn(t + 1)
            _on2(b1, lambda bb: issue_h(t + 1, bb))

        @pl.when(valid(t + 2))
        def _():
            _on3(lax.rem(t + 2, 3), lambda bb: issue_idx(t + 2, bb))

        @pl.when(valid(t))
        def _():
            _on2(b0, wait_gathers)
            _on2(b0, wait_h)
            _on2(b0, combine)
            _on2(b0, lambda bb: issue_store(t, bb))
        return 0

    lax.fori_loop(0, _GC_T, step, 0)

    @pl.when(valid(_GC_T - 1))
    def _():
        wait_store(_GC_T - 1, (_GC_T - 1) % 2)


@functools.partial(
    pl.kernel,
    out_type=jax.ShapeDtypeStruct((E, H), jnp.float32),
    mesh=_MESH,
    scratch_types=(
        [pltpu.VMEM((_GCH, H), jnp.float32)] * 6
        + [pltpu.VMEM((_GCH,), jnp.int32)] * 6
        + [pltpu.SemaphoreType.DMA] * 9
    ),
)
def _sc_gather_combine(h_hbm, a_hbm, b_hbm, srt_hbm, end_hbm, out_hbm, *scr):
    _gather_body(h_hbm, a_hbm, b_hbm, srt_hbm, end_hbm, out_hbm, *scr)


_FCH = 64
_F_NCH = E // _FCH
_F_T = -(-_F_NCH // NW)


def _make_fused(write_out, scat_from_srt):
    def body(h_hbm, a_hbm, b_hbm, srt_hbm, end_hbm, z_hbm, *rest):
        if write_out:
            s_out, h_out = rest[0], rest[1]
            scr = rest[2:]
        else:
            s_out = rest[0]
            h_out = None
            scr = rest[1:]
        acc = scr[0]
        h_rows = scr[1:3]
        a_rows = scr[3:5]
        b_rows = scr[5:7]
        sidx = scr[7:10]
        eidx = scr[10:13]
        sem_h = scr[13:15]
        sem_idx = scr[15:18]
        sem_g = scr[18:20]
        sem_st = scr[20:22]
        sem_sc = scr[22:24]
        scat = sidx if scat_from_srt else eidx

        cid = lax.axis_index("c")
        sid = lax.axis_index("s")
        wid = sid * NC + cid

        @pl.when(sid == 0)
        def _():
            pltpu.sync_copy(z_hbm, acc)

        plsc.subcore_barrier()

        def valid(t):
            return wid + t * NW < _F_NCH

        def chunk_of(t):
            return wid + t * NW

        def issue_idx(t, b):
            c = chunk_of(t)
            pltpu.async_copy(srt_hbm.at[pl.ds(c * _FCH, _FCH)], sidx[b],
                             sem_idx[b])
            pltpu.async_copy(end_hbm.at[pl.ds(c * _FCH, _FCH)], eidx[b],
                             sem_idx[b])

        def wait_idx(b):
            pltpu.make_async_copy(srt_hbm.at[pl.ds(0, _FCH)], sidx[b],
                                  sem_idx[b]).wait()
            pltpu.make_async_copy(srt_hbm.at[pl.ds(0, _FCH)], eidx[b],
                                  sem_idx[b]).wait()

        def issue_gathers(bi, bg):
            pltpu.async_copy(a_hbm.at[sidx[bi]], a_rows[bg], sem_g[bg])
            pltpu.async_copy(b_hbm.at[eidx[bi]], b_rows[bg], sem_g[bg])

        def wait_gathers(b):
            pltpu.make_async_copy(a_hbm.at[pl.ds(0, _FCH)], a_rows[b],
                                  sem_g[b]).wait()
            pltpu.make_async_copy(b_hbm.at[pl.ds(0, _FCH)], b_rows[b],
                                  sem_g[b]).wait()

        def issue_h(t, b):
            pltpu.async_copy(h_hbm.at[pl.ds(chunk_of(t) * _FCH, _FCH)],
                             h_rows[b], sem_h[b])

        def wait_h(b):
            pltpu.make_async_copy(h_hbm.at[pl.ds(0, _FCH)], h_rows[b],
                                  sem_h[b]).wait()

        def issue_store(t, b):
            pltpu.async_copy(h_rows[b],
                             h_out.at[pl.ds(chunk_of(t) * _FCH, _FCH)],
                             sem_st[b])

        def wait_store(b):
            pltpu.make_async_copy(h_rows[b], h_out.at[pl.ds(0, _FCH)],
                                  sem_st[b]).wait()

        def issue_sc(bi, bg):
            for g in range(_FCH // 16):
                vec = scat[bi][pl.ds(g * 16, 16)]
                pltpu.async_copy(h_rows[bg].at[pl.ds(g * 16, 16)],
                                 acc.at[vec], sem_sc[bg], add=True)

        def wait_sc(b):
            for g in range(_FCH // 16):
                pltpu.make_async_copy(h_rows[b].at[pl.ds(0, 16)],
                                      acc.at[pl.ds(0, 16)], sem_sc[b]).wait()

        def _on2(b, fn):
            for bb in range(2):
                @pl.when(b == bb)
                def _(bb=bb):
                    fn(bb)

        def _on3(b, fn):
            for bb in range(3):
                @pl.when(b == bb)
                def _(bb=bb):
                    fn(bb)

        def _on32(bi, bg, fn):
            for i3 in range(3):
                for i2 in range(2):
                    @pl.when((bi == i3) & (bg == i2))
                    def _(i3=i3, i2=i2):
                        fn(i3, i2)

        def combine(b):
            def row(r, _):
                for kk in range(H // 16):
                    sl = pl.ds(kk * 16, 16)
                    v = h_rows[b][r, sl] + a_rows[b][r, sl] \
                        - b_rows[b][r, sl]
                    h_rows[b][r, sl] = jnp.maximum(v, 0.01 * v)
                return 0

            lax.fori_loop(0, _FCH, row, 0)

        @pl.when(valid(0))
        def _():
            issue_idx(0, 0)
            wait_idx(0)
            issue_gathers(0, 0)
            issue_h(0, 0)

        @pl.when(valid(1))
        def _():
            issue_idx(1, 1)

        def step(t, _):
            b0 = lax.rem(t, 2)
            b1 = lax.rem(t + 1, 2)

            @pl.when((t >= 1) & valid(t - 1))
            def _():
                if write_out:
                    _on2(lax.rem(t + 1, 2), wait_store)
                _on2(lax.rem(t + 1, 2), wait_sc)

            @pl.when(valid(t + 1))
            def _():
                _on3(lax.rem(t + 1, 3), wait_idx)
                _on32(lax.rem(t + 1, 3), b1, issue_gathers)
                _on2(b1, lambda bb: issue_h(t + 1, bb))

            @pl.when(valid(t + 2))
            def _():
                _on3(lax.rem(t + 2, 3), lambda bb: issue_idx(t + 2, bb))

            @pl.when(valid(t))
            def _():
                _on2(b0, wait_gathers)
                _on2(b0, wait_h)
                _on2(b0, combine)
                if write_out:
                    _on2(b0, lambda bb: issue_store(t, bb))
                _on32(lax.rem(t, 3), b0, issue_sc)
            return 0

        lax.fori_loop(0, _F_T, step, 0)

        @pl.when(valid(_F_T - 1))
        def _():
            if write_out:
                wait_store((_F_T - 1) % 2)
            wait_sc((_F_T - 1) % 2)

        plsc.subcore_barrier()

        @pl.when(sid == 0)
        def _():
            pltpu.sync_copy(acc, s_out.at[cid])

    if write_out:
        outs = [jax.ShapeDtypeStruct((NC, N, H), jnp.float32),
                jax.ShapeDtypeStruct((E, H), jnp.float32)]
    else:
        outs = jax.ShapeDtypeStruct((NC, N, H), jnp.float32)
    return pl.kernel(
        body,
        out_type=outs,
        mesh=_MESH,
        scratch_types=(
            [pltpu.VMEM_SHARED((N, H), jnp.float32)]
            + [pltpu.VMEM((_FCH, H), jnp.float32)] * 6
            + [pltpu.VMEM((_FCH,), jnp.int32)] * 6
            + [pltpu.SemaphoreType.DMA] * 11
        ),
    )


_FUSED_MID = _make_fused(True, False)
_FUSED_LAST = _make_fused(False, True)


def _final_body(hn_ref, batch_ref, wm_ref, bm_ref, wo_ref, bo_ref,
                ae_ref, out_ref, hg_acc):
    i = pl.program_id(0)
    hn = hn_ref[0] + hn_ref[1]
    ae = jnp.dot(hn, wm_ref[...], preferred_element_type=jnp.float32) \
        + bm_ref[...]
    ae = jnp.maximum(ae, 0.01 * ae)
    ae_ref[...] = ae
    nrm = jnp.sqrt(jnp.sum(ae * ae, axis=1, keepdims=True))
    an = ae / jnp.maximum(nrm, 1e-12)
    bvec = batch_ref[0, 0, :]
    onehot = (bvec[:, None]
              == lax.broadcasted_iota(jnp.int32, (_BN, G), 1)).astype(
                  jnp.float32)
    contrib = lax.dot_general(onehot, an, (((0,), (0,)), ((), ())),
                              preferred_element_type=jnp.float32)

    @pl.when(i == 0)
    def _():
        hg_acc[...] = jnp.zeros_like(hg_acc)

    hg_acc[...] += contrib

    @pl.when(i == N // _BN - 1)
    def _():
        out_ref[...] = jnp.dot(hg_acc[...], wo_ref[...],
                               preferred_element_type=jnp.float32) \
            + bo_ref[...]


def _tc_final(hn_parts, batch3, wm_t, bm_row, wo_t, bo_row):
    return pl.pallas_call(
        _final_body,
        grid=(N // _BN,),
        in_specs=[
            pl.BlockSpec((NC, _BN, H), lambda i: (0, i, 0)),
            pl.BlockSpec((1, 1, _BN), lambda i: (i, 0, 0)),
            pl.BlockSpec((H, H), lambda i: (0, 0)),
            pl.BlockSpec((1, H), lambda i: (0, 0)),
            pl.BlockSpec((H, H), lambda i: (0, 0)),
            pl.BlockSpec((1, H), lambda i: (0, 0)),
        ],
        out_specs=[
            pl.BlockSpec((_BN, H), lambda i: (i, 0)),
            pl.BlockSpec((G, H), lambda i: (0, 0)),
        ],
        out_shape=[
            jax.ShapeDtypeStruct((N, H), jnp.float32),
            jax.ShapeDtypeStruct((G, H), jnp.float32),
        ],
        scratch_shapes=[pltpu.VMEM((G, H), jnp.float32)],
    )(hn_parts, batch3, wm_t, bm_row, wo_t, bo_row)


def kernel(x, concat_feats, srt_concat_batch, end_concat_batch,
           num_concat_feats, batch, W_init, b_init, W_h1, b_h1,
           W_mol, b_mol, W_out, b_out):
    srt1 = srt_concat_batch.astype(jnp.int32)
    end1 = end_concat_batch.astype(jnp.int32)

    zeros_n = jnp.zeros((N_PAD, H), jnp.float32)
    wh1_t = W_h1.T
    bh1_row = b_h1[None, :]

    zeros_acc = jnp.zeros((N, H), jnp.float32)

    h = _tc_h0(concat_feats, W_init.T, b_init[None, :])
    s_parts = _sc_scatter(h, end1, zeros_n)
    for layer in range(3):
        a, b = _tc_ab(s_parts, h, wh1_t, bh1_row)
        if layer < 2:
            s_parts, h = _FUSED_MID(h, a, b, srt1, end1, zeros_acc)
        else:
            hn_parts = _FUSED_LAST(h, a, b, srt1, end1, zeros_acc)
    atom_embs, out = _tc_final(
        hn_parts, batch.astype(jnp.int32).reshape(N // _BN, 1, _BN),
        W_mol.T, b_mol[None, :], W_out.T, b_out[None, :])
    return (out, atom_embs)
